# Initial kernel scaffold; baseline (speedup 1.0000x reference)
import jax
import jax.numpy as jnp
from jax.experimental import pallas as pl


def kernel(x, table, W, b):
    raise NotImplementedError("write your pallas kernel here")


# speedup vs baseline: 30.1414x; 30.1414x over previous
import functools

import jax
import jax.numpy as jnp
from jax import lax
from jax.experimental import pallas as pl
from jax.experimental.pallas import tpu as pltpu
from jax.experimental.pallas import tpu_sc as plsc

V = 100000
D = 128
B = 4096
L = 200

NC = 2
NS = 16
NW = NC * NS
ROWS_PER_W = B // NW
GROUPS = ROWS_PER_W // 16


_VBLK = 5000


def _project_body(tab_ref, w_ref, o_ref):
    o_ref[...] = jnp.sum(tab_ref[...] * w_ref[...], axis=1, keepdims=True)


def _project_table(table, W):
    return pl.pallas_call(
        _project_body,
        grid=(V // _VBLK,),
        in_specs=[
            pl.BlockSpec((_VBLK, D), lambda i: (i, 0)),
            pl.BlockSpec((1, D), lambda i: (0, 0)),
        ],
        out_specs=pl.BlockSpec((_VBLK, 1), lambda i: (i, 0)),
        out_shape=jax.ShapeDtypeStruct((V, 1), jnp.float32),
    )(table, W)


def _pool_body(t_hbm, x_hbm, b_hbm, out_hbm, t_v, x_v, b_v, out_v,
               sem_t, sem_x):
    wid = lax.axis_index("s") * NC + lax.axis_index("c")
    base = wid * ROWS_PER_W

    cp_t = pltpu.make_async_copy(t_hbm, t_v, sem_t)
    cp_t.start()
    cp_x = pltpu.make_async_copy(
        x_hbm.at[pl.ds(base * L, ROWS_PER_W * L)], x_v, sem_x)
    cp_x.start()
    pltpu.sync_copy(b_hbm, b_v)
    cp_x.wait()
    cp_t.wait()

    lane = lax.broadcasted_iota(jnp.int32, (16,), 0)
    zero = jnp.zeros((16,), jnp.float32)
    b_vec = b_v[...]

    for g in range(GROUPS):
        flat0 = (g * 16 + lane) * L

        def body(j, carry):
            acc, cnt = carry
            xi = plsc.load_gather(x_v, [flat0 + j])
            tv = plsc.load_gather(t_v, [xi])
            m = xi != 0
            acc = acc + jnp.where(m, tv, 0.0)
            cnt = cnt + jnp.where(m, 1.0, 0.0)
            return acc, cnt

        acc, cnt = lax.fori_loop(0, L, body, (zero, zero))
        out_v[pl.ds(g * 16, 16)] = acc / jnp.maximum(cnt, 1.0) + b_vec

    pltpu.sync_copy(out_v, out_hbm.at[pl.ds(base, ROWS_PER_W)])


@functools.cache
def _make_pool_kernel():
    mesh = plsc.VectorSubcoreMesh(core_axis_name="c", subcore_axis_name="s")
    return pl.kernel(
        _pool_body,
        out_type=jax.ShapeDtypeStruct((B,), jnp.float32),
        mesh=mesh,
        scratch_types=[
            pltpu.VMEM((V,), jnp.float32),
            pltpu.VMEM((ROWS_PER_W * L,), jnp.int32),
            pltpu.VMEM((16,), jnp.float32),
            pltpu.VMEM((ROWS_PER_W,), jnp.float32),
            pltpu.SemaphoreType.DMA,
            pltpu.SemaphoreType.DMA,
        ],
        compiler_params=pltpu.CompilerParams(needs_layout_passes=False),
    )


def kernel(x, table, W, b):
    t = _project_table(table, W).reshape(V)
    b16 = jnp.broadcast_to(b.reshape(1), (16,))
    return _make_pool_kernel()(t, x.reshape(B * L), b16)


# speedup vs baseline: 35.8581x; 1.1897x over previous
import functools

import jax
import jax.numpy as jnp
from jax import lax
from jax.experimental import pallas as pl
from jax.experimental.pallas import tpu as pltpu
from jax.experimental.pallas import tpu_sc as plsc

V = 100000
D = 128
B = 4096
L = 200

NC = 2
NS = 16
NW = NC * NS
ROWS_PER_W = B // NW
GROUPS = ROWS_PER_W // 16


_VBLK = 4096
_VGRID = -(-V // _VBLK)
_VPAD = _VGRID * _VBLK


def _project_body(tab_ref, w_ref, o_ref):
    i = pl.program_id(0)
    s = jnp.sum(tab_ref[...] * w_ref[...], axis=1)
    o_ref[pl.ds(i * _VBLK, _VBLK)] = s


def _project_table(table, W):
    return pl.pallas_call(
        _project_body,
        grid=(_VGRID,),
        in_specs=[
            pl.BlockSpec((_VBLK, D), lambda i: (i, 0)),
            pl.BlockSpec((1, D), lambda i: (0, 0)),
        ],
        out_specs=pl.BlockSpec((_VPAD,), lambda i: (0,)),
        out_shape=jax.ShapeDtypeStruct((_VPAD,), jnp.float32),
    )(table, W)


def _pool_body(t_hbm, x_hbm, b_hbm, out_hbm, t_v, x_v, b_v, out_v,
               sem_t, sem_x):
    wid = lax.axis_index("s") * NC + lax.axis_index("c")
    base = wid * ROWS_PER_W

    cp_t = pltpu.make_async_copy(t_hbm, t_v, sem_t)
    cp_t.start()
    cp_x = pltpu.make_async_copy(
        x_hbm.at[pl.ds(base * L, ROWS_PER_W * L)], x_v, sem_x)
    cp_x.start()
    pltpu.sync_copy(b_hbm, b_v)
    cp_x.wait()
    cp_t.wait()

    lane = lax.broadcasted_iota(jnp.int32, (16,), 0)
    zero = jnp.zeros((16,), jnp.float32)
    b_vec = b_v[...]

    for g in range(GROUPS):
        flat0 = (g * 16 + lane) * L

        def body(j, carry):
            acc, cnt = carry
            xi = plsc.load_gather(x_v, [flat0 + j])
            tv = plsc.load_gather(t_v, [xi])
            m = xi != 0
            acc = acc + jnp.where(m, tv, 0.0)
            cnt = cnt + jnp.where(m, 1.0, 0.0)
            return acc, cnt

        acc, cnt = lax.fori_loop(0, L, body, (zero, zero), unroll=8)
        out_v[pl.ds(g * 16, 16)] = acc / jnp.maximum(cnt, 1.0) + b_vec

    pltpu.sync_copy(out_v, out_hbm.at[pl.ds(base, ROWS_PER_W)])


@functools.cache
def _make_pool_kernel():
    mesh = plsc.VectorSubcoreMesh(core_axis_name="c", subcore_axis_name="s")
    return pl.kernel(
        _pool_body,
        out_type=jax.ShapeDtypeStruct((B,), jnp.float32),
        mesh=mesh,
        scratch_types=[
            pltpu.VMEM((_VPAD,), jnp.float32),
            pltpu.VMEM((ROWS_PER_W * L,), jnp.int32),
            pltpu.VMEM((16,), jnp.float32),
            pltpu.VMEM((ROWS_PER_W,), jnp.float32),
            pltpu.SemaphoreType.DMA,
            pltpu.SemaphoreType.DMA,
        ],
        compiler_params=pltpu.CompilerParams(needs_layout_passes=False),
    )


def kernel(x, table, W, b):
    t = _project_table(table, W)
    b16 = jnp.broadcast_to(b.reshape(1), (16,))
    return _make_pool_kernel()(t, x.reshape(B * L), b16)
